# baseline (speedup 1.0000x reference)
import functools

import jax
import jax.numpy as jnp
from jax import lax
from jax.experimental import pallas as pl
from jax.experimental.pallas import tpu as pltpu
from jax.experimental.pallas import tpu_sc as plsc

N = 10000
NPAD = 10240
E = 160000
EPAD = 163840
F = 256
FH = 128
CH = 128
NS = 16
NC = 2
ROWS_PER_TILE = NPAD // NS
A_CHUNKS = EPAD // 32 // CH
GC = 64
GCHUNKS = EPAD // NS // GC
PASSES = 4
GPP = GCHUNKS // PASSES

_mesh = plsc.VectorSubcoreMesh(core_axis_name="c", subcore_axis_name="s")


@functools.partial(
    pl.kernel,
    out_type=jax.ShapeDtypeStruct((NC, NPAD), jnp.float32),
    mesh=_mesh,
    scratch_types=[
        pltpu.VMEM((A_CHUNKS, CH), jnp.int32),
        pltpu.VMEM((CH,), jnp.float32),
        pltpu.VMEM_SHARED((NPAD,), jnp.float32),
    ],
)
def _deg_kernel(col4_hbm, ones_hbm, zeros_hbm, degp_hbm, colbuf, onesbuf, dacc):
    c = lax.axis_index("c")
    s = lax.axis_index("s")
    wid = c * NS + s
    pltpu.sync_copy(zeros_hbm, dacc.at[pl.ds(s * ROWS_PER_TILE, ROWS_PER_TILE)])
    pltpu.sync_copy(ones_hbm, onesbuf)
    pltpu.sync_copy(col4_hbm.at[wid], colbuf)
    plsc.subcore_barrier()

    def body(j, carry):
        pltpu.sync_copy(onesbuf, dacc.at[colbuf.at[j]], add=True)
        return carry

    lax.fori_loop(0, A_CHUNKS, body, 0)
    plsc.subcore_barrier()
    sl = pl.ds(s * ROWS_PER_TILE, ROWS_PER_TILE)
    pltpu.sync_copy(dacc.at[sl], degp_hbm.at[c, sl])


@functools.partial(
    pl.kernel,
    out_type=jax.ShapeDtypeStruct((NC, NPAD, FH), jnp.float32),
    mesh=_mesh,
    scratch_types=[
        pltpu.VMEM((GPP + 8, GC), jnp.int32),
        pltpu.VMEM((GPP, GC), jnp.int32),
        pltpu.VMEM((GC, FH), jnp.float32),
        pltpu.VMEM((GC, FH), jnp.float32),
        pltpu.VMEM((GC, FH), jnp.float32),
        pltpu.VMEM((GC, FH), jnp.float32),
        pltpu.VMEM_SHARED((NPAD, FH), jnp.float32),
        pltpu.SemaphoreType.DMA,
        pltpu.SemaphoreType.DMA,
        pltpu.SemaphoreType.DMA,
        pltpu.SemaphoreType.DMA,
    ],
)
def _scatter_kernel(hs2_hbm, rowA_hbm, rowB_hbm, col3_hbm, out3_hbm,
                    rbuf, cbuf, buf0, buf1, buf2, buf3, acc,
                    sem0, sem1, sem2, sem3):
    c = lax.axis_index("c")
    s = lax.axis_index("s")
    bufs = (buf0, buf1, buf2, buf3)
    sems = (sem0, sem1, sem2, sem3)
    sl = pl.ds(s * ROWS_PER_TILE, ROWS_PER_TILE)
    pltpu.sync_copy(hs2_hbm.at[pl.ds(c * NPAD + s * ROWS_PER_TILE,
                                     ROWS_PER_TILE)], acc.at[sl])
    plsc.subcore_barrier()

    def gissue(j, k):
        pltpu.async_copy(hs2_hbm.at[rbuf.at[j]], bufs[k], sems[k])

    def gwait(k):
        pltpu.make_async_copy(hs2_hbm.at[rbuf.at[0]], bufs[k], sems[k]).wait()

    for h in range(PASSES):
        @pl.when(c == 0)
        def _():
            pltpu.sync_copy(rowA_hbm.at[s, pl.ds(h * GPP, GPP + 8)], rbuf)

        @pl.when(c == 1)
        def _():
            pltpu.sync_copy(rowB_hbm.at[s, pl.ds(h * GPP, GPP + 8)], rbuf)

        pltpu.sync_copy(col3_hbm.at[s, pl.ds(h * GPP, GPP)], cbuf)
        for k in range(4):
            gissue(k, k)

        def body(jj, carry):
            j = 4 * jj
            for k in range(4):
                gwait(k)
                pltpu.sync_copy(bufs[k], acc.at[cbuf.at[j + k]], add=True)
                gissue(j + k + 4, k)
            return carry

        lax.fori_loop(0, GPP // 4, body, 0)
        for k in range(4):
            gwait(k)

    plsc.subcore_barrier()
    pltpu.sync_copy(acc.at[sl], out3_hbm.at[c, sl])


def _tc1_body(x_ref, w_ref, degp_ref, out_ref):
    h = jnp.dot(x_ref[...], w_ref[...], preferred_element_type=jnp.float32)
    d = degp_ref[0] + degp_ref[1] + 1.0
    dinv = lax.rsqrt(d)[:, None]
    out_ref[0] = h[:, :FH] * dinv
    out_ref[1] = h[:, FH:] * dinv


def _tc1(xpad, W, degp):
    return pl.pallas_call(
        _tc1_body,
        grid=(NPAD // 256,),
        in_specs=[
            pl.BlockSpec((256, F), lambda i: (i, 0)),
            pl.BlockSpec((F, F), lambda i: (0, 0)),
            pl.BlockSpec((NC, 256), lambda i: (0, i)),
        ],
        out_specs=pl.BlockSpec((NC, 256, FH), lambda i: (0, i, 0)),
        out_shape=jax.ShapeDtypeStruct((NC, NPAD, FH), jnp.float32),
    )(xpad, W, degp)


def _tc2_body(acc3_ref, degp_ref, b_ref, out_ref):
    d = degp_ref[0] + degp_ref[1] + 1.0
    dinv = lax.rsqrt(d)[:, None]
    b = b_ref[...]
    out_ref[:, :FH] = acc3_ref[0] * dinv + b[:FH][None, :]
    out_ref[:, FH:] = acc3_ref[1] * dinv + b[FH:][None, :]


def _tc2(acc3, degp, b):
    return pl.pallas_call(
        _tc2_body,
        grid=(NPAD // 128,),
        in_specs=[
            pl.BlockSpec((NC, 128, FH), lambda i: (0, i, 0)),
            pl.BlockSpec((NC, 128), lambda i: (0, i)),
            pl.BlockSpec((F,), lambda i: (0,)),
        ],
        out_specs=pl.BlockSpec((128, F), lambda i: (i, 0)),
        out_shape=jax.ShapeDtypeStruct((NPAD, F), jnp.float32),
    )(acc3, degp, b)


def kernel(x, edge_index, W, b):
    row = edge_index[0]
    col = edge_index[1]
    pad = jnp.full((EPAD - E,), NPAD - 1, dtype=jnp.int32)
    rowp = jnp.concatenate([row, pad])
    colp = jnp.concatenate([col, pad])
    col4 = colp.reshape(32, A_CHUNKS, CH)
    rowA = jnp.concatenate(
        [rowp.reshape(NS, GCHUNKS, GC),
         jnp.full((NS, 8, GC), NPAD - 1, dtype=jnp.int32)], axis=1)
    rowB = rowA + jnp.int32(NPAD)
    col3 = colp.reshape(NS, GCHUNKS, GC)
    xpad = jnp.zeros((NPAD, F), jnp.float32).at[:N].set(x)
    ones128 = jnp.ones((CH,), jnp.float32)
    zeros640 = jnp.zeros((ROWS_PER_TILE,), jnp.float32)

    degp = _deg_kernel(col4, ones128, zeros640)
    hs3 = _tc1(xpad, W, degp)
    hs2 = hs3.reshape(NC * NPAD, FH)
    acc3 = _scatter_kernel(hs2, rowA, rowB, col3)
    return _tc2(acc3, degp, b)[:N]


# speedup vs baseline: 1.0752x; 1.0752x over previous
import functools

import jax
import jax.numpy as jnp
from jax import lax
from jax.experimental import pallas as pl
from jax.experimental.pallas import tpu as pltpu
from jax.experimental.pallas import tpu_sc as plsc

N = 10000
NPAD = 10240
E = 160000
EPAD = 163840
F = 256
FH = 128
CH = 128
NS = 16
NC = 2
ROWS_PER_TILE = NPAD // NS
A_CHUNKS = EPAD // 32 // CH
GC = 128
GCHUNKS = EPAD // NS // GC
PASSES = 2
GPP = GCHUNKS // PASSES

_mesh = plsc.VectorSubcoreMesh(core_axis_name="c", subcore_axis_name="s")


@functools.partial(
    pl.kernel,
    out_type=jax.ShapeDtypeStruct((NC, NPAD), jnp.float32),
    mesh=_mesh,
    scratch_types=[
        pltpu.VMEM((A_CHUNKS, CH), jnp.int32),
        pltpu.VMEM((CH,), jnp.float32),
        pltpu.VMEM_SHARED((NPAD,), jnp.float32),
    ],
)
def _deg_kernel(col4_hbm, ones_hbm, zeros_hbm, degp_hbm, colbuf, onesbuf, dacc):
    c = lax.axis_index("c")
    s = lax.axis_index("s")
    wid = c * NS + s
    pltpu.sync_copy(zeros_hbm, dacc.at[pl.ds(s * ROWS_PER_TILE, ROWS_PER_TILE)])
    pltpu.sync_copy(ones_hbm, onesbuf)
    pltpu.sync_copy(col4_hbm.at[wid], colbuf)
    plsc.subcore_barrier()

    def body(j, carry):
        pltpu.sync_copy(onesbuf, dacc.at[colbuf.at[j]], add=True)
        return carry

    lax.fori_loop(0, A_CHUNKS, body, 0)
    plsc.subcore_barrier()
    sl = pl.ds(s * ROWS_PER_TILE, ROWS_PER_TILE)
    pltpu.sync_copy(dacc.at[sl], degp_hbm.at[c, sl])


@functools.partial(
    pl.kernel,
    out_type=jax.ShapeDtypeStruct((NC, NPAD, FH), jnp.float32),
    mesh=_mesh,
    scratch_types=[
        pltpu.VMEM((GPP + 8, GC), jnp.int32),
        pltpu.VMEM((GPP, GC), jnp.int32),
        pltpu.VMEM((GC, FH), jnp.float32),
        pltpu.VMEM((GC, FH), jnp.float32),
        pltpu.VMEM_SHARED((NPAD, FH), jnp.float32),
        pltpu.SemaphoreType.DMA,
        pltpu.SemaphoreType.DMA,
    ],
)
def _scatter_kernel(hs2_hbm, rowA_hbm, rowB_hbm, col3_hbm, out3_hbm,
                    rbuf, cbuf, buf0, buf1, acc, sem0, sem1):
    c = lax.axis_index("c")
    s = lax.axis_index("s")
    sl = pl.ds(s * ROWS_PER_TILE, ROWS_PER_TILE)
    pltpu.sync_copy(hs2_hbm.at[pl.ds(c * NPAD + s * ROWS_PER_TILE,
                                     ROWS_PER_TILE)], acc.at[sl])
    plsc.subcore_barrier()

    def gissue(j, buf, sem):
        pltpu.async_copy(hs2_hbm.at[rbuf.at[j]], buf, sem)

    def gwait(buf, sem):
        pltpu.make_async_copy(hs2_hbm.at[rbuf.at[0]], buf, sem).wait()

    for h in range(PASSES):
        @pl.when(c == 0)
        def _():
            pltpu.sync_copy(rowA_hbm.at[s, pl.ds(h * GPP, GPP + 8)], rbuf)

        @pl.when(c == 1)
        def _():
            pltpu.sync_copy(rowB_hbm.at[s, pl.ds(h * GPP, GPP + 8)], rbuf)

        pltpu.sync_copy(col3_hbm.at[s, pl.ds(h * GPP, GPP)], cbuf)
        gissue(0, buf0, sem0)

        def body(jj, carry):
            j = 2 * jj
            gwait(buf0, sem0)
            gissue(j + 1, buf1, sem1)
            pltpu.sync_copy(buf0, acc.at[cbuf.at[j]], add=True)
            gwait(buf1, sem1)
            gissue(j + 2, buf0, sem0)
            pltpu.sync_copy(buf1, acc.at[cbuf.at[j + 1]], add=True)
            return carry

        lax.fori_loop(0, GPP // 2, body, 0)
        gwait(buf0, sem0)

    plsc.subcore_barrier()
    pltpu.sync_copy(acc.at[sl], out3_hbm.at[c, sl])


def _tc1_body(x_ref, w_ref, degp_ref, out_ref):
    h = jnp.dot(x_ref[...], w_ref[...], preferred_element_type=jnp.float32)
    d = degp_ref[0] + degp_ref[1] + 1.0
    dinv = lax.rsqrt(d)
    out_ref[0] = h[:, :FH] * dinv
    out_ref[1] = h[:, FH:] * dinv


def _tc1(x, W, degp3):
    return pl.pallas_call(
        _tc1_body,
        grid=(N // 80,),
        in_specs=[
            pl.BlockSpec((80, F), lambda i: (i, 0)),
            pl.BlockSpec((F, F), lambda i: (0, 0)),
            pl.BlockSpec((NC, 80, 1), lambda i: (0, i, 0)),
        ],
        out_specs=pl.BlockSpec((NC, 80, FH), lambda i: (0, i, 0)),
        out_shape=jax.ShapeDtypeStruct((NC, NPAD, FH), jnp.float32),
    )(x, W, degp3)


def _tc2_body(acc3_ref, degp_ref, b_ref, out_ref):
    d = degp_ref[0] + degp_ref[1] + 1.0
    dinv = lax.rsqrt(d)
    b = b_ref[...]
    out_ref[:, :FH] = acc3_ref[0] * dinv + b[:FH][None, :]
    out_ref[:, FH:] = acc3_ref[1] * dinv + b[FH:][None, :]


def _tc2(acc3, degp3, b):
    return pl.pallas_call(
        _tc2_body,
        grid=(N // 80,),
        in_specs=[
            pl.BlockSpec((NC, 80, FH), lambda i: (0, i, 0)),
            pl.BlockSpec((NC, 80, 1), lambda i: (0, i, 0)),
            pl.BlockSpec((F,), lambda i: (0,)),
        ],
        out_specs=pl.BlockSpec((80, F), lambda i: (i, 0)),
        out_shape=jax.ShapeDtypeStruct((N, F), jnp.float32),
    )(acc3, degp3, b)


def kernel(x, edge_index, W, b):
    row = edge_index[0]
    col = edge_index[1]
    pad = jnp.full((EPAD - E,), NPAD - 1, dtype=jnp.int32)
    rowp = jnp.concatenate([row, pad])
    colp = jnp.concatenate([col, pad])
    col4 = colp.reshape(32, A_CHUNKS, CH)
    rowA = jnp.concatenate(
        [rowp.reshape(NS, GCHUNKS, GC),
         jnp.full((NS, 8, GC), NPAD - 1, dtype=jnp.int32)], axis=1)
    rowB = rowA + jnp.int32(NPAD)
    col3 = colp.reshape(NS, GCHUNKS, GC)
    ones128 = jnp.ones((CH,), jnp.float32)
    zeros640 = jnp.zeros((ROWS_PER_TILE,), jnp.float32)

    degp3 = _deg_kernel(col4, ones128, zeros640).reshape(NC, NPAD, 1)
    hs3 = _tc1(x, W, degp3)
    hs2 = hs3.reshape(NC * NPAD, FH)
    acc3 = _scatter_kernel(hs2, rowA, rowB, col3)
    return _tc2(acc3, degp3, b)


# speedup vs baseline: 1.2262x; 1.1405x over previous
import functools

import jax
import jax.numpy as jnp
from jax import lax
from jax.experimental import pallas as pl
from jax.experimental.pallas import tpu as pltpu
from jax.experimental.pallas import tpu_sc as plsc

N = 10000
NPAD = 10240
E = 160000
EPAD = 163840
F = 256
FH = 128
CH = 128
NS = 16
NC = 2
ROWS_PER_TILE = NPAD // NS
A_CHUNKS = EPAD // 32 // CH
GC = 128
GCHUNKS = EPAD // NS // GC
PASSES = 2
GPP = GCHUNKS // PASSES

_mesh = plsc.VectorSubcoreMesh(core_axis_name="c", subcore_axis_name="s")


@functools.partial(
    pl.kernel,
    out_type=jax.ShapeDtypeStruct((NC, NPAD), jnp.float32),
    mesh=_mesh,
    scratch_types=[
        pltpu.VMEM((A_CHUNKS, CH), jnp.int32),
        pltpu.VMEM((CH,), jnp.float32),
        pltpu.VMEM_SHARED((NPAD,), jnp.float32),
    ],
)
def _deg_kernel(col4_hbm, ones_hbm, zeros_hbm, degp_hbm, colbuf, onesbuf, dacc):
    c = lax.axis_index("c")
    s = lax.axis_index("s")
    wid = c * NS + s
    pltpu.sync_copy(zeros_hbm, dacc.at[pl.ds(s * ROWS_PER_TILE, ROWS_PER_TILE)])
    pltpu.sync_copy(ones_hbm, onesbuf)
    pltpu.sync_copy(col4_hbm.at[wid], colbuf)
    plsc.subcore_barrier()

    def body(j, carry):
        pltpu.sync_copy(onesbuf, dacc.at[colbuf.at[j]], add=True)
        return carry

    lax.fori_loop(0, A_CHUNKS, body, 0)
    plsc.subcore_barrier()
    sl = pl.ds(s * ROWS_PER_TILE, ROWS_PER_TILE)
    pltpu.sync_copy(dacc.at[sl], degp_hbm.at[c, sl])


@functools.partial(
    pl.kernel,
    out_type=jax.ShapeDtypeStruct((NC, NPAD, FH), jnp.float32),
    mesh=_mesh,
    scratch_types=[
        pltpu.VMEM((GPP + 8, GC), jnp.int32),
        pltpu.VMEM((GPP, GC), jnp.int32),
        pltpu.VMEM((GC, FH), jnp.float32),
        pltpu.VMEM((GC, FH), jnp.float32),
        pltpu.VMEM_SHARED((NPAD, FH), jnp.float32),
        pltpu.SemaphoreType.DMA,
        pltpu.SemaphoreType.DMA,
    ],
)
def _scatter_kernel(hs2_hbm, rowA_hbm, rowB_hbm, col3_hbm, out3_hbm,
                    rbuf, cbuf, buf0, buf1, acc, sem0, sem1):
    c = lax.axis_index("c")
    s = lax.axis_index("s")
    sl = pl.ds(s * ROWS_PER_TILE, ROWS_PER_TILE)
    pltpu.sync_copy(hs2_hbm.at[pl.ds(c * NPAD + s * ROWS_PER_TILE,
                                     ROWS_PER_TILE)], acc.at[sl])
    plsc.subcore_barrier()

    def gissue(j, buf, sem):
        pltpu.async_copy(hs2_hbm.at[rbuf.at[j]], buf, sem)

    def gwait(buf, sem):
        pltpu.make_async_copy(hs2_hbm.at[rbuf.at[0]], buf, sem).wait()

    for h in range(PASSES):
        @pl.when(c == 0)
        def _():
            pltpu.sync_copy(rowA_hbm.at[s, pl.ds(h * GPP, GPP + 8)], rbuf)

        @pl.when(c == 1)
        def _():
            pltpu.sync_copy(rowB_hbm.at[s, pl.ds(h * GPP, GPP + 8)], rbuf)

        pltpu.sync_copy(col3_hbm.at[s, pl.ds(h * GPP, GPP)], cbuf)
        gissue(0, buf0, sem0)

        def body(jj, carry):
            j = 2 * jj
            gwait(buf0, sem0)
            gissue(j + 1, buf1, sem1)
            pltpu.sync_copy(buf0, acc.at[cbuf.at[j]], add=True)
            gwait(buf1, sem1)
            gissue(j + 2, buf0, sem0)
            pltpu.sync_copy(buf1, acc.at[cbuf.at[j + 1]], add=True)
            return carry

        lax.fori_loop(0, GPP // 2, body, 0)
        gwait(buf0, sem0)

    plsc.subcore_barrier()
    pltpu.sync_copy(acc.at[sl], out3_hbm.at[c, sl])


def _tc1_body(x_ref, w_ref, degp_ref, out_ref):
    h = jnp.dot(x_ref[...], w_ref[...], preferred_element_type=jnp.float32)
    d = degp_ref[0] + degp_ref[1] + 1.0
    dinv = lax.rsqrt(d)[:, None]
    out_ref[0] = h[:, :FH] * dinv
    out_ref[1] = h[:, FH:] * dinv


def _tc1(xpad, W, degp):
    return pl.pallas_call(
        _tc1_body,
        grid=(NPAD // 256,),
        in_specs=[
            pl.BlockSpec((256, F), lambda i: (i, 0)),
            pl.BlockSpec((F, F), lambda i: (0, 0)),
            pl.BlockSpec((NC, 256), lambda i: (0, i)),
        ],
        out_specs=pl.BlockSpec((NC, 256, FH), lambda i: (0, i, 0)),
        out_shape=jax.ShapeDtypeStruct((NC, NPAD, FH), jnp.float32),
    )(xpad, W, degp)


def _tc2_body(acc3_ref, degp_ref, b_ref, out_ref):
    d = degp_ref[0] + degp_ref[1] + 1.0
    dinv = lax.rsqrt(d)[:, None]
    b = b_ref[...]
    out_ref[:, :FH] = acc3_ref[0] * dinv + b[:FH][None, :]
    out_ref[:, FH:] = acc3_ref[1] * dinv + b[FH:][None, :]


def _tc2(acc3, degp, b):
    return pl.pallas_call(
        _tc2_body,
        grid=(NPAD // 128,),
        in_specs=[
            pl.BlockSpec((NC, 128, FH), lambda i: (0, i, 0)),
            pl.BlockSpec((NC, 128), lambda i: (0, i)),
            pl.BlockSpec((F,), lambda i: (0,)),
        ],
        out_specs=pl.BlockSpec((128, F), lambda i: (i, 0)),
        out_shape=jax.ShapeDtypeStruct((NPAD, F), jnp.float32),
    )(acc3, degp, b)


def kernel(x, edge_index, W, b):
    row = edge_index[0]
    col = edge_index[1]
    pad = jnp.full((EPAD - E,), NPAD - 1, dtype=jnp.int32)
    rowp = jnp.concatenate([row, pad])
    colp = jnp.concatenate([col, pad])
    col4 = colp.reshape(32, A_CHUNKS, CH)
    rowA = jnp.concatenate(
        [rowp.reshape(NS, GCHUNKS, GC),
         jnp.full((NS, 8, GC), NPAD - 1, dtype=jnp.int32)], axis=1)
    rowB = rowA + jnp.int32(NPAD)
    col3 = colp.reshape(NS, GCHUNKS, GC)
    xpad = jnp.zeros((NPAD, F), jnp.float32).at[:N].set(x)
    ones128 = jnp.ones((CH,), jnp.float32)
    zeros640 = jnp.zeros((ROWS_PER_TILE,), jnp.float32)

    degp = _deg_kernel(col4, ones128, zeros640)
    hs3 = _tc1(xpad, W, degp)
    hs2 = hs3.reshape(NC * NPAD, FH)
    acc3 = _scatter_kernel(hs2, rowA, rowB, col3)
    return _tc2(acc3, degp, b)[:N]
